# baseline (speedup 1.0000x reference)
import math

import jax
import jax.numpy as jnp
from jax.experimental import pallas as pl
from jax.experimental.pallas import tpu as pltpu

B = 1024
C = 100000
SCALE = 64.0
MARGIN = 0.4
H = 0.333
EPS = 0.001

COL_BLOCK = 3584


def _scale_block(logits_ref, out_ref):
    out_ref[...] = logits_ref[...] * SCALE


def _margins_block(norms_ref, gang_ref, gadd_ref):
    norms = norms_ref[...]
    safe = jnp.clip(norms, 0.001, 100.0)
    mean = jnp.sum(safe) * (1.0 / B)
    var = jnp.sum((safe - mean) ** 2) * (1.0 / (B - 1))
    std = jnp.sqrt(var)
    ms = jnp.clip((safe - mean) / (std + EPS) * H, -1.0, 1.0)
    gang_ref[...] = -MARGIN * ms
    gadd_ref[...] = MARGIN + MARGIN * ms


def _fixup_block(labels_sref, scaled_ref, logits_ref, labels_ref, gang_ref,
                 gadd_ref, out_ref):
    i = pl.program_id(0)
    cb = labels_sref[i] // 128
    local = labels_ref[...] - cb * 128
    cols = jax.lax.broadcasted_iota(jnp.int32, (8, 128), 1)
    valid = (local >= 0) & (local < 128)
    m = (cols == local) & valid

    x = logits_ref[...]
    t = jnp.sum(jnp.where(m, x, 0.0), axis=1, keepdims=True)
    xt = jnp.clip(t, -1.0 + 1e-7, 1.0 - 1e-7)
    g_ang = gang_ref[...]
    g_add = gadd_ref[...]
    cg = jnp.cos(g_ang)
    sg = jnp.sin(g_ang)
    cos_tm = xt * cg - jnp.sqrt(1.0 - xt * xt) * sg
    low = (g_ang < EPS) & (xt > jnp.cos(EPS - g_ang))
    high = (g_ang > -EPS) & (xt < jnp.cos(math.pi - EPS - g_ang))
    cos_eps = math.cos(EPS)
    cos_tm = jnp.where(low, cos_eps, jnp.where(high, -cos_eps, cos_tm))
    fix = (cos_tm - g_add) * SCALE

    out_ref[...] = jnp.where(m, fix, scaled_ref[...])


def kernel(logits, norms, labels):
    labels2d = labels.reshape(B, 1)

    scaled = pl.pallas_call(
        _scale_block,
        grid=(pl.cdiv(C, COL_BLOCK),),
        in_specs=[pl.BlockSpec((B, COL_BLOCK), lambda j: (0, j))],
        out_specs=pl.BlockSpec((B, COL_BLOCK), lambda j: (0, j)),
        out_shape=jax.ShapeDtypeStruct((B, C), jnp.float32),
        compiler_params=pltpu.CompilerParams(
            dimension_semantics=("arbitrary",),
        ),
    )(logits)

    g_ang, g_add = pl.pallas_call(
        _margins_block,
        out_shape=[
            jax.ShapeDtypeStruct((B, 1), jnp.float32),
            jax.ShapeDtypeStruct((B, 1), jnp.float32),
        ],
    )(norms)

    win = lambda i, lab: (i // 8, lab[i] // 128)
    col0 = lambda i, lab: (i // 8, 0)
    out = pl.pallas_call(
        _fixup_block,
        grid_spec=pltpu.PrefetchScalarGridSpec(
            num_scalar_prefetch=1,
            grid=(B,),
            in_specs=[
                pl.BlockSpec((8, 128), win),
                pl.BlockSpec((8, 128), win),
                pl.BlockSpec((8, 1), col0),
                pl.BlockSpec((8, 1), col0),
                pl.BlockSpec((8, 1), col0),
            ],
            out_specs=pl.BlockSpec((8, 128), win),
        ),
        out_shape=jax.ShapeDtypeStruct((B, C), jnp.float32),
        input_output_aliases={1: 0},
        compiler_params=pltpu.CompilerParams(
            dimension_semantics=("arbitrary",),
        ),
    )(labels, scaled, logits, labels2d, g_ang, g_add)
    return out


# speedup vs baseline: 1.4780x; 1.4780x over previous
import math

import jax
import jax.numpy as jnp
from jax.experimental import pallas as pl
from jax.experimental.pallas import tpu as pltpu

B = 1024
C = 100000
SCALE = 64.0
MARGIN = 0.4
H = 0.333
EPS = 0.001

COL_BLOCK = 2816


def _adaface_block(logits_ref, norms_ref, labels_ref, out_ref):
    j = pl.program_id(0)
    x = logits_ref[...]
    labels = labels_ref[...]
    norms = norms_ref[...]

    safe = jnp.clip(norms, 0.001, 100.0)
    mean = jnp.sum(safe) * (1.0 / B)
    var = jnp.sum((safe - mean) ** 2) * (1.0 / (B - 1))
    std = jnp.sqrt(var)
    ms = jnp.clip((safe - mean) / (std + EPS) * H, -1.0, 1.0)
    g_ang = -MARGIN * ms
    g_add = MARGIN + MARGIN * ms

    col0 = j * COL_BLOCK
    cols = col0 + jax.lax.broadcasted_iota(jnp.int32, (B, COL_BLOCK), 1)
    mask = cols == labels

    t = jnp.sum(jnp.where(mask, x, 0.0), axis=1, keepdims=True)
    xt = jnp.clip(t, -1.0 + 1e-7, 1.0 - 1e-7)
    cg = jnp.cos(g_ang)
    sg = jnp.sin(g_ang)
    cos_tm = xt * cg - jnp.sqrt(1.0 - xt * xt) * sg
    low = (g_ang < EPS) & (xt > jnp.cos(EPS - g_ang))
    high = (g_ang > -EPS) & (xt < jnp.cos(math.pi - EPS - g_ang))
    cos_eps = math.cos(EPS)
    cos_tm = jnp.where(low, cos_eps, jnp.where(high, -cos_eps, cos_tm))
    fix = (cos_tm - g_add) * SCALE

    out_ref[...] = jnp.where(mask, fix, x * SCALE)


def kernel(logits, norms, labels):
    num_blocks = pl.cdiv(C, COL_BLOCK)
    labels2d = labels.reshape(B, 1)
    return pl.pallas_call(
        _adaface_block,
        grid=(num_blocks,),
        in_specs=[
            pl.BlockSpec((B, COL_BLOCK), lambda j: (0, j)),
            pl.BlockSpec((B, 1), lambda j: (0, 0)),
            pl.BlockSpec((B, 1), lambda j: (0, 0)),
        ],
        out_specs=pl.BlockSpec((B, COL_BLOCK), lambda j: (0, j)),
        out_shape=jax.ShapeDtypeStruct((B, C), jnp.float32),
        compiler_params=pltpu.CompilerParams(
            dimension_semantics=("arbitrary",),
        ),
    )(logits, norms, labels2d)
